# Initial kernel scaffold; baseline (speedup 1.0000x reference)
import jax
import jax.numpy as jnp
from jax.experimental import pallas as pl


def kernel(x_node, edge_index, W, b):
    raise NotImplementedError("write your pallas kernel here")


# speedup vs baseline: 6.8989x; 6.8989x over previous
import functools

import jax
import jax.numpy as jnp
from jax import lax
from jax.experimental import pallas as pl
from jax.experimental.pallas import tpu as pltpu
from jax.experimental.pallas import tpu_sc as plsc

NC = 2
NS = 16
L = 16
NW = NC * NS

C = 32
MBUF = 4
NBUF = 2

N = 10000
D = 128
NPAD = 10240
YROWS = 10112
RPT = YROWS // NS

CN = 64


def _mv_body(x_hbm, rows_hbm, cols_hbm, norm_hbm, out_hbm,
             rb, cb, nb, gb, sb, y_sh, msems, gsems, ssems,
             *, nchunk):
  cid = lax.axis_index("c")
  sid = lax.axis_index("s")
  wid = cid * NS + sid
  ebase = wid * (nchunk * C)

  zv = jnp.zeros((L,), jnp.float32)

  def zrow(r, _):
    for j in range(D // L):
      sb[0][r, pl.ds(j * L, L)] = zv
    return 0

  lax.fori_loop(0, C, zrow, 0)
  base = sid * RPT
  for k in range(RPT // C):
    pltpu.sync_copy(sb[0], y_sh.at[pl.ds(base + k * C, C)])
  rem = RPT % C
  if rem:
    pltpu.sync_copy(sb[0].at[pl.ds(0, rem)],
                    y_sh.at[pl.ds(base + (RPT // C) * C, rem)])
  plsc.subcore_barrier()

  def meta_issue(i, m):
    off = ebase + i * C
    pltpu.async_copy(rows_hbm.at[pl.ds(off, C)], rb[m], msems[m])
    pltpu.async_copy(cols_hbm.at[pl.ds(off, C)], cb[m], msems[m])
    pltpu.async_copy(norm_hbm.at[pl.ds(off, C)], nb[m], msems[m])

  def meta_wait(m):
    pltpu.make_async_copy(rows_hbm.at[pl.ds(0, C)], rb[m], msems[m]).wait()
    pltpu.make_async_copy(cols_hbm.at[pl.ds(0, C)], cb[m], msems[m]).wait()
    pltpu.make_async_copy(norm_hbm.at[pl.ds(0, C)], nb[m], msems[m]).wait()

  def gather_issue(m, b):
    pltpu.async_copy(x_hbm.at[rb[m]], gb[b], gsems[b])

  def gather_wait(b):
    pltpu.make_async_copy(x_hbm.at[rb[0]], gb[b], gsems[b]).wait()

  def scatter_issue(m, b):
    pltpu.async_copy(sb[b], y_sh.at[cb[m]], ssems[b], add=True)

  def scatter_wait(b):
    pltpu.make_async_copy(sb[0], y_sh.at[cb[0]], ssems[b]).wait()

  def scale(m, b):
    def qbody(q, _):
      nv = nb[m][pl.ds(q * L, L)]
      e0 = q * L
      for t in range(L):
        sv = jnp.full((L,), nv[t], jnp.float32)
        for j in range(D // L):
          sb[b][e0 + t, pl.ds(j * L, L)] = gb[b][e0 + t, pl.ds(j * L, L)] * sv
      return 0
    lax.fori_loop(0, C // L, qbody, 0)

  meta_issue(0, 0)
  meta_issue(1, 1)
  meta_wait(0)
  gather_issue(0, 0)
  gather_wait(0)
  meta_issue(2, 2)
  scale(0, 0)
  scatter_issue(0, 0)
  meta_wait(1)
  gather_issue(1, 1)
  gather_wait(1)
  meta_issue(3, 3)
  scale(1, 1)
  scatter_issue(1, 1)
  meta_wait(2)
  gather_issue(2, 0)

  def group(g, _):
    for j in range(4):
      i = (g - 1) * 4 + 2 + j
      b = j % 2
      m = (2 + j) % 4
      gather_wait(b)
      scatter_wait(b)
      meta_issue(i + 2, j % 4)
      scale(m, b)
      scatter_issue(m, b)
      meta_wait((3 + j) % 4)
      gather_issue((3 + j) % 4, 1 - b)
    return 0

  lax.fori_loop(1, (nchunk - 4) // 4 + 1, group, 0)

  gather_wait(0)
  scatter_wait(0)
  scale(2, 0)
  scatter_issue(2, 0)
  meta_wait(3)
  gather_issue(3, 1)
  gather_wait(1)
  scatter_wait(1)
  scale(3, 1)
  scatter_issue(3, 1)
  scatter_wait(0)
  scatter_wait(1)
  plsc.subcore_barrier()

  pltpu.sync_copy(y_sh.at[pl.ds(sid * RPT, RPT)],
                  out_hbm.at[cid, pl.ds(sid * RPT, RPT)])


def _norm_body(rows_hbm, cols_hbm, norm_hbm,
               rows_v, cols_v, wbuf, deg_v, dinv_v, deg_sh,
               *, nchunk):
  cid = lax.axis_index("c")
  sid = lax.axis_index("s")
  wid = cid * NS + sid
  tpe = nchunk * CN

  seg = NPAD // NS
  zv = jnp.zeros((L,), jnp.float32)

  def zdeg(i, _):
    deg_v[pl.ds(i * L, L)] = zv
    return 0

  lax.fori_loop(0, NPAD // L, zdeg, 0)
  pltpu.sync_copy(deg_v.at[pl.ds(0, seg)], deg_sh.at[pl.ds(sid * seg, seg)])
  plsc.subcore_barrier()

  for t in range(2):
    wt = sid * 2 + t
    pltpu.sync_copy(rows_hbm.at[wt], rows_v)
    pltpu.sync_copy(cols_hbm.at[wt], cols_v)

    def wcomp(ci, _):
      for k in range(CN // L):
        r = rows_v[ci, pl.ds(k * L, L)]
        c = cols_v[ci, pl.ds(k * L, L)]
        wbuf[ci, pl.ds(k * L, L)] = jnp.where(r != c, 1.0, 0.0)
      return 0

    lax.fori_loop(0, nchunk, wcomp, 0)

    def wscat(ci, _):
      pltpu.sync_copy(wbuf.at[ci], deg_sh.at[rows_v.at[ci]], add=True)
      return 0

    lax.fori_loop(0, nchunk, wscat, 0)

  plsc.subcore_barrier()

  pltpu.sync_copy(deg_sh, deg_v)

  def rsq(i, _):
    d = deg_v[pl.ds(i * L, L)]
    ds = jnp.maximum(d, 1.0)
    y = jnp.full((L,), 0.015625, jnp.float32)
    for _ in range(16):
      y = y * (1.5 - 0.5 * ds * y * y)
    dinv_v[pl.ds(i * L, L)] = jnp.where(d > 0.5, y, 0.0)
    return 0

  lax.fori_loop(0, NPAD // L, rsq, 0)

  pltpu.sync_copy(rows_hbm.at[wid], rows_v)
  pltpu.sync_copy(cols_hbm.at[wid], cols_v)

  def ncomp(ci, _):
    for k in range(CN // L):
      r = rows_v[ci, pl.ds(k * L, L)]
      c = cols_v[ci, pl.ds(k * L, L)]
      dr = plsc.load_gather(dinv_v, [r])
      dc = plsc.load_gather(dinv_v, [c])
      wbuf[ci, pl.ds(k * L, L)] = jnp.where(r != c, -(dr * dc), 0.0)
    return 0

  lax.fori_loop(0, nchunk, ncomp, 0)

  def nout(ci, _):
    pltpu.sync_copy(wbuf.at[ci], norm_hbm.at[pl.ds(wid * tpe + ci * CN, CN)])
    return 0

  lax.fori_loop(0, nchunk, nout, 0)


def _make_sc_kernels(nchunk, nchunk_n, e_pad):
  mesh = plsc.VectorSubcoreMesh(core_axis_name="c", subcore_axis_name="s")
  params = pltpu.CompilerParams(needs_layout_passes=False)

  mv = pl.kernel(
      functools.partial(_mv_body, nchunk=nchunk),
      out_type=jax.ShapeDtypeStruct((NC, YROWS, D), jnp.float32),
      mesh=mesh,
      scratch_types=[
          [pltpu.VMEM((C,), jnp.int32) for _ in range(MBUF)],
          [pltpu.VMEM((C,), jnp.int32) for _ in range(MBUF)],
          [pltpu.VMEM((C,), jnp.float32) for _ in range(MBUF)],
          [pltpu.VMEM((C, D), jnp.float32) for _ in range(NBUF)],
          [pltpu.VMEM((C, D), jnp.float32) for _ in range(NBUF)],
          pltpu.VMEM_SHARED((YROWS, D), jnp.float32),
          [pltpu.SemaphoreType.DMA for _ in range(MBUF)],
          [pltpu.SemaphoreType.DMA for _ in range(NBUF)],
          [pltpu.SemaphoreType.DMA for _ in range(NBUF)],
      ],
      compiler_params=params,
  )

  norm_k = pl.kernel(
      functools.partial(_norm_body, nchunk=nchunk_n),
      out_type=jax.ShapeDtypeStruct((e_pad,), jnp.float32),
      mesh=mesh,
      scratch_types=[
          pltpu.VMEM((nchunk_n, CN), jnp.int32),
          pltpu.VMEM((nchunk_n, CN), jnp.int32),
          pltpu.VMEM((nchunk_n, CN), jnp.float32),
          pltpu.VMEM((NPAD,), jnp.float32),
          pltpu.VMEM((NPAD,), jnp.float32),
          pltpu.VMEM_SHARED((NPAD,), jnp.float32),
      ],
      compiler_params=params,
  )
  return mv, norm_k


RBLK = 400


def _init_body(x_ref, w0_ref, b_ref, out_ref):
  out_ref[...] = (
      jnp.dot(x_ref[...], w0_ref[...], preferred_element_type=jnp.float32)
      + b_ref[...])


def _init(x, w0, b2d):
  return pl.pallas_call(
      _init_body,
      grid=(N // RBLK,),
      in_specs=[
          pl.BlockSpec((RBLK, D), lambda i: (i, 0)),
          pl.BlockSpec((D, D), lambda i: (0, 0)),
          pl.BlockSpec((1, D), lambda i: (0, 0)),
      ],
      out_specs=pl.BlockSpec((RBLK, D), lambda i: (i, 0)),
      out_shape=jax.ShapeDtypeStruct((N, D), jnp.float32),
  )(x, w0, b2d)


def _comb_body(yp_ref, prev_ref, acc_ref, wk_ref, ab_ref, tx_ref, out_ref):
  a = ab_ref[0, 0]
  be = ab_ref[0, 1]
  tx = a * (yp_ref[0] + yp_ref[1]) - be * prev_ref[...]
  tx_ref[...] = tx
  out_ref[...] = acc_ref[...] + jnp.dot(
      tx, wk_ref[0], preferred_element_type=jnp.float32)


def _comb(yp, prev, acc, wk, ab):
  return pl.pallas_call(
      _comb_body,
      grid=(N // RBLK,),
      in_specs=[
          pl.BlockSpec((NC, RBLK, D), lambda i: (0, i, 0)),
          pl.BlockSpec((RBLK, D), lambda i: (i, 0)),
          pl.BlockSpec((RBLK, D), lambda i: (i, 0)),
          pl.BlockSpec((1, D, D), lambda i: (0, 0, 0)),
          pl.BlockSpec((1, 2), lambda i: (0, 0)),
      ],
      out_specs=[
          pl.BlockSpec((RBLK, D), lambda i: (i, 0)),
          pl.BlockSpec((RBLK, D), lambda i: (i, 0)),
      ],
      out_shape=[
          jax.ShapeDtypeStruct((N, D), jnp.float32),
          jax.ShapeDtypeStruct((N, D), jnp.float32),
      ],
  )(yp, prev, acc, wk, ab)


def kernel(x_node, edge_index, W, b):
  n, d = x_node.shape
  e = edge_index.shape[0]
  assert (n, d) == (N, D)

  tpe = -(-e // NW)
  tpe = -(-tpe // 128) * 128
  nchunk = tpe // C
  nchunk_n = tpe // CN
  e_pad = tpe * NW

  row = edge_index[:, 0]
  col = edge_index[:, 1]
  pad = jnp.arange(e_pad - e, dtype=jnp.int32) % N
  rows1 = jnp.concatenate([row, pad])
  cols1 = jnp.concatenate([col, pad])
  rows3 = rows1.reshape(NW, nchunk_n, CN)
  cols3 = cols1.reshape(NW, nchunk_n, CN)

  mv, norm_k = _make_sc_kernels(nchunk, nchunk_n, e_pad)

  norm1 = norm_k(rows3, cols3)
  b2d = b.reshape(1, D)
  out0 = _init(x_node, W[0], b2d)

  def step(carry, i):
    txm1, txm2, out = carry
    yp = mv(txm1, rows1, cols1, norm1)
    a = jnp.where(i == 0, 1.0, 2.0)
    be = jnp.where(i == 0, 0.0, 1.0)
    ab = jnp.stack([a, be]).reshape(1, 2).astype(jnp.float32)
    wk = lax.dynamic_slice_in_dim(W, i + 1, 1, axis=0)
    tx, out = _comb(yp, txm2, out, wk, ab)
    return (tx, txm1, out), 0

  (_, _, out), _ = lax.scan(step, (x_node, x_node, out0),
                            jnp.arange(4, dtype=jnp.int32))
  return out


# speedup vs baseline: 9.3990x; 1.3624x over previous
import functools

import jax
import jax.numpy as jnp
from jax import lax
from jax.experimental import pallas as pl
from jax.experimental.pallas import tpu as pltpu
from jax.experimental.pallas import tpu_sc as plsc

NC = 2
NS = 16
L = 16
NW = NC * NS

C = 32
MBUF = 4
NBUF = 2

N = 10000
D = 128
NPAD = 10240
YROWS = 10112
RPT = YROWS // NS

CN = 64


def _mv_body(x_hbm, rows_hbm, cols_hbm, norm_hbm, out_hbm,
             rb, cb, nb, gb, sb, y_sh, msems, gsems, ssems,
             *, nchunk):
  cid = lax.axis_index("c")
  sid = lax.axis_index("s")
  wid = cid * NS + sid
  ebase = wid * (nchunk * C)

  zv = jnp.zeros((L,), jnp.float32)

  def zrow(r, _):
    for j in range(D // L):
      sb[0][r, pl.ds(j * L, L)] = zv
    return 0

  lax.fori_loop(0, C, zrow, 0)
  base = sid * RPT
  for k in range(RPT // C):
    pltpu.sync_copy(sb[0], y_sh.at[pl.ds(base + k * C, C)])
  rem = RPT % C
  if rem:
    pltpu.sync_copy(sb[0].at[pl.ds(0, rem)],
                    y_sh.at[pl.ds(base + (RPT // C) * C, rem)])
  plsc.subcore_barrier()

  def meta_issue(i, m):
    off = ebase + i * C
    pltpu.async_copy(rows_hbm.at[pl.ds(off, C)], rb[m], msems[m])
    pltpu.async_copy(cols_hbm.at[pl.ds(off, C)], cb[m], msems[m])
    pltpu.async_copy(norm_hbm.at[pl.ds(off, C)], nb[m], msems[m])

  def meta_wait(m):
    pltpu.make_async_copy(rows_hbm.at[pl.ds(0, C)], rb[m], msems[m]).wait()
    pltpu.make_async_copy(cols_hbm.at[pl.ds(0, C)], cb[m], msems[m]).wait()
    pltpu.make_async_copy(norm_hbm.at[pl.ds(0, C)], nb[m], msems[m]).wait()

  def gather_issue(m, b):
    pltpu.async_copy(x_hbm.at[rb[m]], gb[b], gsems[b])

  def gather_wait(b):
    pltpu.make_async_copy(x_hbm.at[rb[0]], gb[b], gsems[b]).wait()

  def scatter_issue(m, b):
    pltpu.async_copy(sb[b], y_sh.at[cb[m]], ssems[b], add=True)

  def scatter_wait(b):
    pltpu.make_async_copy(sb[0], y_sh.at[cb[0]], ssems[b]).wait()

  def scale(m, b):
    for q in range(C // L):
      nv = nb[m][pl.ds(q * L, L)]
      for t in range(L):
        sv = jnp.full((L,), nv[t], jnp.float32)
        e0 = q * L + t
        for j in range(D // L):
          sb[b][e0, pl.ds(j * L, L)] = gb[b][e0, pl.ds(j * L, L)] * sv

  meta_issue(0, 0)
  meta_issue(1, 1)
  meta_wait(0)
  gather_issue(0, 0)
  meta_wait(1)
  gather_issue(1, 1)
  gather_wait(0)
  meta_issue(2, 2)
  scale(0, 0)
  scatter_issue(0, 0)
  meta_wait(2)
  gather_issue(2, 0)
  gather_wait(1)
  meta_issue(3, 3)
  scale(1, 1)
  scatter_issue(1, 1)

  def group(g, _):
    for j in range(4):
      i = (g - 1) * 4 + 2 + j
      b = j % 2
      m = (2 + j) % 4
      meta_wait((3 + j) % 4)
      gather_issue((3 + j) % 4, 1 - b)
      gather_wait(b)
      scatter_wait(b)
      meta_issue(i + 2, j % 4)
      scale(m, b)
      scatter_issue(m, b)
    return 0

  lax.fori_loop(1, (nchunk - 4) // 4 + 1, group, 0)

  meta_wait(3)
  gather_issue(3, 1)
  gather_wait(0)
  scatter_wait(0)
  scale(2, 0)
  scatter_issue(2, 0)
  gather_wait(1)
  scatter_wait(1)
  scale(3, 1)
  scatter_issue(3, 1)
  scatter_wait(0)
  scatter_wait(1)
  plsc.subcore_barrier()

  pltpu.sync_copy(y_sh.at[pl.ds(sid * RPT, RPT)],
                  out_hbm.at[cid, pl.ds(sid * RPT, RPT)])


def _norm_body(rows_hbm, cols_hbm, norm_hbm,
               rows_v, cols_v, wbuf, deg_v, dinv_v, deg_sh,
               *, nchunk):
  cid = lax.axis_index("c")
  sid = lax.axis_index("s")
  wid = cid * NS + sid
  tpe = nchunk * CN

  seg = NPAD // NS
  zv = jnp.zeros((L,), jnp.float32)

  def zdeg(i, _):
    deg_v[pl.ds(i * L, L)] = zv
    return 0

  lax.fori_loop(0, NPAD // L, zdeg, 0)
  pltpu.sync_copy(deg_v.at[pl.ds(0, seg)], deg_sh.at[pl.ds(sid * seg, seg)])
  plsc.subcore_barrier()

  for t in range(2):
    wt = sid * 2 + t
    pltpu.sync_copy(rows_hbm.at[wt], rows_v)
    pltpu.sync_copy(cols_hbm.at[wt], cols_v)

    def wcomp(ci, _):
      for k in range(CN // L):
        r = rows_v[ci, pl.ds(k * L, L)]
        c = cols_v[ci, pl.ds(k * L, L)]
        wbuf[ci, pl.ds(k * L, L)] = jnp.where(r != c, 1.0, 0.0)
      return 0

    lax.fori_loop(0, nchunk, wcomp, 0)

    def wscat(ci, _):
      pltpu.sync_copy(wbuf.at[ci], deg_sh.at[rows_v.at[ci]], add=True)
      return 0

    lax.fori_loop(0, nchunk, wscat, 0)

  plsc.subcore_barrier()

  pltpu.sync_copy(deg_sh, deg_v)

  def rsq(i, _):
    d = deg_v[pl.ds(i * L, L)]
    ds = jnp.maximum(d, 1.0)
    y = jnp.full((L,), 0.015625, jnp.float32)
    for _ in range(16):
      y = y * (1.5 - 0.5 * ds * y * y)
    dinv_v[pl.ds(i * L, L)] = jnp.where(d > 0.5, y, 0.0)
    return 0

  lax.fori_loop(0, NPAD // L, rsq, 0)

  pltpu.sync_copy(rows_hbm.at[wid], rows_v)
  pltpu.sync_copy(cols_hbm.at[wid], cols_v)

  def ncomp(ci, _):
    for k in range(CN // L):
      r = rows_v[ci, pl.ds(k * L, L)]
      c = cols_v[ci, pl.ds(k * L, L)]
      dr = plsc.load_gather(dinv_v, [r])
      dc = plsc.load_gather(dinv_v, [c])
      wbuf[ci, pl.ds(k * L, L)] = jnp.where(r != c, -(dr * dc), 0.0)
    return 0

  lax.fori_loop(0, nchunk, ncomp, 0)

  def nout(ci, _):
    pltpu.sync_copy(wbuf.at[ci], norm_hbm.at[pl.ds(wid * tpe + ci * CN, CN)])
    return 0

  lax.fori_loop(0, nchunk, nout, 0)


def _make_sc_kernels(nchunk, nchunk_n, e_pad):
  mesh = plsc.VectorSubcoreMesh(core_axis_name="c", subcore_axis_name="s")
  params = pltpu.CompilerParams(needs_layout_passes=False)

  mv = pl.kernel(
      functools.partial(_mv_body, nchunk=nchunk),
      out_type=jax.ShapeDtypeStruct((NC, YROWS, D), jnp.float32),
      mesh=mesh,
      scratch_types=[
          [pltpu.VMEM((C,), jnp.int32) for _ in range(MBUF)],
          [pltpu.VMEM((C,), jnp.int32) for _ in range(MBUF)],
          [pltpu.VMEM((C,), jnp.float32) for _ in range(MBUF)],
          [pltpu.VMEM((C, D), jnp.float32) for _ in range(NBUF)],
          [pltpu.VMEM((C, D), jnp.float32) for _ in range(NBUF)],
          pltpu.VMEM_SHARED((YROWS, D), jnp.float32),
          [pltpu.SemaphoreType.DMA for _ in range(MBUF)],
          [pltpu.SemaphoreType.DMA for _ in range(NBUF)],
          [pltpu.SemaphoreType.DMA for _ in range(NBUF)],
      ],
      compiler_params=params,
  )

  norm_k = pl.kernel(
      functools.partial(_norm_body, nchunk=nchunk_n),
      out_type=jax.ShapeDtypeStruct((e_pad,), jnp.float32),
      mesh=mesh,
      scratch_types=[
          pltpu.VMEM((nchunk_n, CN), jnp.int32),
          pltpu.VMEM((nchunk_n, CN), jnp.int32),
          pltpu.VMEM((nchunk_n, CN), jnp.float32),
          pltpu.VMEM((NPAD,), jnp.float32),
          pltpu.VMEM((NPAD,), jnp.float32),
          pltpu.VMEM_SHARED((NPAD,), jnp.float32),
      ],
      compiler_params=params,
  )
  return mv, norm_k


RBLK = 400


def _init_body(x_ref, w0_ref, b_ref, out_ref):
  out_ref[...] = (
      jnp.dot(x_ref[...], w0_ref[...], preferred_element_type=jnp.float32)
      + b_ref[...])


def _init(x, w0, b2d):
  return pl.pallas_call(
      _init_body,
      grid=(N // RBLK,),
      in_specs=[
          pl.BlockSpec((RBLK, D), lambda i: (i, 0)),
          pl.BlockSpec((D, D), lambda i: (0, 0)),
          pl.BlockSpec((1, D), lambda i: (0, 0)),
      ],
      out_specs=pl.BlockSpec((RBLK, D), lambda i: (i, 0)),
      out_shape=jax.ShapeDtypeStruct((N, D), jnp.float32),
  )(x, w0, b2d)


def _comb_body(yp_ref, prev_ref, acc_ref, wk_ref, ab_ref, tx_ref, out_ref):
  a = ab_ref[0, 0]
  be = ab_ref[0, 1]
  tx = a * (yp_ref[0] + yp_ref[1]) - be * prev_ref[...]
  tx_ref[...] = tx
  out_ref[...] = acc_ref[...] + jnp.dot(
      tx, wk_ref[0], preferred_element_type=jnp.float32)


def _comb(yp, prev, acc, wk, ab):
  return pl.pallas_call(
      _comb_body,
      grid=(N // RBLK,),
      in_specs=[
          pl.BlockSpec((NC, RBLK, D), lambda i: (0, i, 0)),
          pl.BlockSpec((RBLK, D), lambda i: (i, 0)),
          pl.BlockSpec((RBLK, D), lambda i: (i, 0)),
          pl.BlockSpec((1, D, D), lambda i: (0, 0, 0)),
          pl.BlockSpec((1, 2), lambda i: (0, 0)),
      ],
      out_specs=[
          pl.BlockSpec((RBLK, D), lambda i: (i, 0)),
          pl.BlockSpec((RBLK, D), lambda i: (i, 0)),
      ],
      out_shape=[
          jax.ShapeDtypeStruct((N, D), jnp.float32),
          jax.ShapeDtypeStruct((N, D), jnp.float32),
      ],
  )(yp, prev, acc, wk, ab)


def kernel(x_node, edge_index, W, b):
  n, d = x_node.shape
  e = edge_index.shape[0]
  assert (n, d) == (N, D)

  tpe = -(-e // NW)
  tpe = -(-tpe // 128) * 128
  nchunk = tpe // C
  nchunk_n = tpe // CN
  e_pad = tpe * NW

  row = edge_index[:, 0]
  col = edge_index[:, 1]
  pad = jnp.arange(e_pad - e, dtype=jnp.int32) % N
  rows1 = jnp.concatenate([row, pad])
  cols1 = jnp.concatenate([col, pad])
  rows3 = rows1.reshape(NW, nchunk_n, CN)
  cols3 = cols1.reshape(NW, nchunk_n, CN)

  mv, norm_k = _make_sc_kernels(nchunk, nchunk_n, e_pad)

  norm1 = norm_k(rows3, cols3)
  b2d = b.reshape(1, D)
  out0 = _init(x_node, W[0], b2d)

  def step(carry, i):
    txm1, txm2, out = carry
    yp = mv(txm1, rows1, cols1, norm1)
    a = jnp.where(i == 0, 1.0, 2.0)
    be = jnp.where(i == 0, 0.0, 1.0)
    ab = jnp.stack([a, be]).reshape(1, 2).astype(jnp.float32)
    wk = lax.dynamic_slice_in_dim(W, i + 1, 1, axis=0)
    tx, out = _comb(yp, txm2, out, wk, ab)
    return (tx, txm1, out), 0

  (_, _, out), _ = lax.scan(step, (x_node, x_node, out0),
                            jnp.arange(4, dtype=jnp.int32))
  return out


# speedup vs baseline: 9.7003x; 1.0321x over previous
import functools

import jax
import jax.numpy as jnp
from jax import lax
from jax.experimental import pallas as pl
from jax.experimental.pallas import tpu as pltpu
from jax.experimental.pallas import tpu_sc as plsc

NC = 2
NS = 16
L = 16
NW = NC * NS

C = 32
MBUF = 4
NBUF = 2

N = 10000
D = 128
NPAD = 10240
YROWS = 10112
RPT = YROWS // NS

CN = 64


def _mv_body(x_hbm, rows_hbm, cols_hbm, norm_hbm, out_hbm,
             rb, cb, nb, gb, sb, y_sh, msems, gsems, ssems,
             *, nchunk):
  cid = lax.axis_index("c")
  sid = lax.axis_index("s")
  wid = cid * NS + sid
  ebase = wid * (nchunk * C)

  zv = jnp.zeros((L,), jnp.float32)

  def zrow(r, _):
    for j in range(D // L):
      sb[0][r, pl.ds(j * L, L)] = zv
    return 0

  lax.fori_loop(0, C, zrow, 0)
  base = sid * RPT
  for k in range(RPT // C):
    pltpu.sync_copy(sb[0], y_sh.at[pl.ds(base + k * C, C)])
  rem = RPT % C
  if rem:
    pltpu.sync_copy(sb[0].at[pl.ds(0, rem)],
                    y_sh.at[pl.ds(base + (RPT // C) * C, rem)])
  plsc.subcore_barrier()

  def meta_issue(i, m):
    off = ebase + i * C
    pltpu.async_copy(rows_hbm.at[pl.ds(off, C)], rb[m], msems[m])
    pltpu.async_copy(cols_hbm.at[pl.ds(off, C)], cb[m], msems[m])
    pltpu.async_copy(norm_hbm.at[pl.ds(off, C)], nb[m], msems[m])

  def meta_wait(m):
    pltpu.make_async_copy(rows_hbm.at[pl.ds(0, C)], rb[m], msems[m]).wait()
    pltpu.make_async_copy(cols_hbm.at[pl.ds(0, C)], cb[m], msems[m]).wait()
    pltpu.make_async_copy(norm_hbm.at[pl.ds(0, C)], nb[m], msems[m]).wait()

  def gather_issue(m, b):
    pltpu.async_copy(x_hbm.at[rb[m]], gb[b], gsems[b])

  def gather_wait(b):
    pltpu.make_async_copy(x_hbm.at[rb[0]], gb[b], gsems[b]).wait()

  def scatter_issue(m, b):
    pltpu.async_copy(sb[b], y_sh.at[cb[m]], ssems[b], add=True)

  def scatter_wait(b):
    pltpu.make_async_copy(sb[0], y_sh.at[cb[0]], ssems[b]).wait()

  def scale(m, b):
    for q in range(C // L):
      nv = nb[m][pl.ds(q * L, L)]
      for t in range(L):
        sv = jnp.full((L,), nv[t], jnp.float32)
        e0 = q * L + t
        for j in range(D // L):
          sb[b][e0, pl.ds(j * L, L)] = gb[b][e0, pl.ds(j * L, L)] * sv

  meta_issue(0, 0)
  meta_issue(1, 1)
  meta_wait(0)
  gather_issue(0, 0)
  meta_wait(1)
  gather_issue(1, 1)
  gather_wait(0)
  meta_issue(2, 2)
  scale(0, 0)
  scatter_issue(0, 0)
  meta_wait(2)
  gather_issue(2, 0)
  gather_wait(1)
  meta_issue(3, 3)
  scale(1, 1)
  scatter_issue(1, 1)

  def group(g, _):
    for j in range(4):
      i = (g - 1) * 4 + 2 + j
      b = j % 2
      m = (2 + j) % 4
      meta_wait((3 + j) % 4)
      gather_issue((3 + j) % 4, 1 - b)
      gather_wait(b)
      scatter_wait(b)
      meta_issue(i + 2, j % 4)
      scale(m, b)
      scatter_issue(m, b)
    return 0

  lax.fori_loop(1, (nchunk - 4) // 4 + 1, group, 0)

  meta_wait(3)
  gather_issue(3, 1)
  gather_wait(0)
  scatter_wait(0)
  scale(2, 0)
  scatter_issue(2, 0)
  gather_wait(1)
  scatter_wait(1)
  scale(3, 1)
  scatter_issue(3, 1)
  scatter_wait(0)
  scatter_wait(1)
  plsc.subcore_barrier()

  pltpu.sync_copy(y_sh.at[pl.ds(sid * RPT, RPT)],
                  out_hbm.at[cid, pl.ds(sid * RPT, RPT)])


def _norm_body(rows_hbm, cols_hbm, norm_hbm,
               rows_v, cols_v, wbuf, deg_v, dinv_v, deg_sh, dinv_sh, dsem,
               *, nchunk):
  cid = lax.axis_index("c")
  sid = lax.axis_index("s")
  wid = cid * NS + sid
  tpe = nchunk * CN
  seg = NPAD // NS

  zv = jnp.zeros((L,), jnp.float32)

  def zdeg(i, _):
    deg_v[pl.ds(i * L, L)] = zv
    return 0

  lax.fori_loop(0, NPAD // L, zdeg, 0)
  pltpu.sync_copy(deg_v.at[pl.ds(0, seg)], deg_sh.at[pl.ds(sid * seg, seg)])
  plsc.subcore_barrier()

  for t in range(2):
    wt = sid * 2 + t
    pltpu.sync_copy(rows_hbm.at[wt], rows_v)
    pltpu.sync_copy(cols_hbm.at[wt], cols_v)

    def wrow(ci, _):
      for k in range(CN // L):
        r = rows_v[ci, pl.ds(k * L, L)]
        c = cols_v[ci, pl.ds(k * L, L)]
        wbuf[ci, pl.ds(k * L, L)] = jnp.where(r != c, 1.0, 0.0)
      return 0

    lax.fori_loop(0, nchunk, wrow, 0)

    def wscat(ci, _):
      pltpu.async_copy(wbuf.at[ci], deg_sh.at[rows_v.at[ci]], dsem, add=True)
      return 0

    lax.fori_loop(0, nchunk, wscat, 0)

    def wdrain(ci, _):
      pltpu.make_async_copy(wbuf.at[0], deg_sh.at[rows_v.at[0]], dsem).wait()
      return 0

    lax.fori_loop(0, nchunk, wdrain, 0)

  plsc.subcore_barrier()

  pltpu.sync_copy(deg_sh.at[pl.ds(sid * seg, seg)], deg_v.at[pl.ds(0, seg)])

  def rsq(i, _):
    d = deg_v[pl.ds(i * L, L)]
    ds = jnp.maximum(d, 1.0)
    y = jnp.full((L,), 0.015625, jnp.float32)
    for _ in range(16):
      y = y * (1.5 - 0.5 * ds * y * y)
    dinv_v[pl.ds(i * L, L)] = jnp.where(d > 0.5, y, 0.0)
    return 0

  lax.fori_loop(0, seg // L, rsq, 0)
  pltpu.sync_copy(dinv_v.at[pl.ds(0, seg)], dinv_sh.at[pl.ds(sid * seg, seg)])
  plsc.subcore_barrier()
  pltpu.sync_copy(dinv_sh, dinv_v)

  pltpu.sync_copy(rows_hbm.at[wid], rows_v)
  pltpu.sync_copy(cols_hbm.at[wid], cols_v)

  def ncomp(ci, _):
    for k in range(CN // L):
      r = rows_v[ci, pl.ds(k * L, L)]
      c = cols_v[ci, pl.ds(k * L, L)]
      dr = plsc.load_gather(dinv_v, [r])
      dc = plsc.load_gather(dinv_v, [c])
      wbuf[ci, pl.ds(k * L, L)] = jnp.where(r != c, -(dr * dc), 0.0)
    return 0

  lax.fori_loop(0, nchunk, ncomp, 0)

  def nout(ci, _):
    pltpu.async_copy(wbuf.at[ci], norm_hbm.at[pl.ds(wid * tpe + ci * CN, CN)],
                     dsem)
    return 0

  lax.fori_loop(0, nchunk, nout, 0)

  def ndrain(ci, _):
    pltpu.make_async_copy(wbuf.at[0],
                          norm_hbm.at[pl.ds(0, CN)], dsem).wait()
    return 0

  lax.fori_loop(0, nchunk, ndrain, 0)


def _make_sc_kernels(nchunk, nchunk_n, e_pad):
  mesh = plsc.VectorSubcoreMesh(core_axis_name="c", subcore_axis_name="s")
  params = pltpu.CompilerParams(needs_layout_passes=False)

  mv = pl.kernel(
      functools.partial(_mv_body, nchunk=nchunk),
      out_type=jax.ShapeDtypeStruct((NC, YROWS, D), jnp.float32),
      mesh=mesh,
      scratch_types=[
          [pltpu.VMEM((C,), jnp.int32) for _ in range(MBUF)],
          [pltpu.VMEM((C,), jnp.int32) for _ in range(MBUF)],
          [pltpu.VMEM((C,), jnp.float32) for _ in range(MBUF)],
          [pltpu.VMEM((C, D), jnp.float32) for _ in range(NBUF)],
          [pltpu.VMEM((C, D), jnp.float32) for _ in range(NBUF)],
          pltpu.VMEM_SHARED((YROWS, D), jnp.float32),
          [pltpu.SemaphoreType.DMA for _ in range(MBUF)],
          [pltpu.SemaphoreType.DMA for _ in range(NBUF)],
          [pltpu.SemaphoreType.DMA for _ in range(NBUF)],
      ],
      compiler_params=params,
  )

  norm_k = pl.kernel(
      functools.partial(_norm_body, nchunk=nchunk_n),
      out_type=jax.ShapeDtypeStruct((e_pad,), jnp.float32),
      mesh=mesh,
      scratch_types=[
          pltpu.VMEM((nchunk_n, CN), jnp.int32),
          pltpu.VMEM((nchunk_n, CN), jnp.int32),
          pltpu.VMEM((nchunk_n, CN), jnp.float32),
          pltpu.VMEM((NPAD,), jnp.float32),
          pltpu.VMEM((NPAD,), jnp.float32),
          pltpu.VMEM_SHARED((NPAD,), jnp.float32),
          pltpu.VMEM_SHARED((NPAD,), jnp.float32),
          pltpu.SemaphoreType.DMA,
      ],
      compiler_params=params,
  )
  return mv, norm_k


RBLK = 400


def _init_body(x_ref, w0_ref, b_ref, out_ref):
  out_ref[...] = (
      jnp.dot(x_ref[...], w0_ref[...], preferred_element_type=jnp.float32)
      + b_ref[...])


def _init(x, w0, b2d):
  return pl.pallas_call(
      _init_body,
      grid=(N // RBLK,),
      in_specs=[
          pl.BlockSpec((RBLK, D), lambda i: (i, 0)),
          pl.BlockSpec((D, D), lambda i: (0, 0)),
          pl.BlockSpec((1, D), lambda i: (0, 0)),
      ],
      out_specs=pl.BlockSpec((RBLK, D), lambda i: (i, 0)),
      out_shape=jax.ShapeDtypeStruct((N, D), jnp.float32),
  )(x, w0, b2d)


def _comb_body(yp_ref, prev_ref, acc_ref, wk_ref, ab_ref, tx_ref, out_ref):
  a = ab_ref[0, 0]
  be = ab_ref[0, 1]
  tx = a * (yp_ref[0] + yp_ref[1]) - be * prev_ref[...]
  tx_ref[...] = tx
  out_ref[...] = acc_ref[...] + jnp.dot(
      tx, wk_ref[0], preferred_element_type=jnp.float32)


def _comb(yp, prev, acc, wk, ab):
  return pl.pallas_call(
      _comb_body,
      grid=(N // RBLK,),
      in_specs=[
          pl.BlockSpec((NC, RBLK, D), lambda i: (0, i, 0)),
          pl.BlockSpec((RBLK, D), lambda i: (i, 0)),
          pl.BlockSpec((RBLK, D), lambda i: (i, 0)),
          pl.BlockSpec((1, D, D), lambda i: (0, 0, 0)),
          pl.BlockSpec((1, 2), lambda i: (0, 0)),
      ],
      out_specs=[
          pl.BlockSpec((RBLK, D), lambda i: (i, 0)),
          pl.BlockSpec((RBLK, D), lambda i: (i, 0)),
      ],
      out_shape=[
          jax.ShapeDtypeStruct((N, D), jnp.float32),
          jax.ShapeDtypeStruct((N, D), jnp.float32),
      ],
  )(yp, prev, acc, wk, ab)


def kernel(x_node, edge_index, W, b):
  n, d = x_node.shape
  e = edge_index.shape[0]
  assert (n, d) == (N, D)

  tpe = -(-e // NW)
  tpe = -(-tpe // 128) * 128
  nchunk = tpe // C
  nchunk_n = tpe // CN
  e_pad = tpe * NW

  row = edge_index[:, 0]
  col = edge_index[:, 1]
  pad = jnp.arange(e_pad - e, dtype=jnp.int32) % N
  rows1 = jnp.concatenate([row, pad])
  cols1 = jnp.concatenate([col, pad])
  rows3 = rows1.reshape(NW, nchunk_n, CN)
  cols3 = cols1.reshape(NW, nchunk_n, CN)

  mv, norm_k = _make_sc_kernels(nchunk, nchunk_n, e_pad)

  norm1 = norm_k(rows3, cols3)
  b2d = b.reshape(1, D)
  out0 = _init(x_node, W[0], b2d)

  def step(carry, i):
    txm1, txm2, out = carry
    yp = mv(txm1, rows1, cols1, norm1)
    a = jnp.where(i == 0, 1.0, 2.0)
    be = jnp.where(i == 0, 0.0, 1.0)
    ab = jnp.stack([a, be]).reshape(1, 2).astype(jnp.float32)
    wk = lax.dynamic_slice_in_dim(W, i + 1, 1, axis=0)
    tx, out = _comb(yp, txm2, out, wk, ab)
    return (tx, txm1, out), 0

  (_, _, out), _ = lax.scan(step, (x_node, x_node, out0),
                            jnp.arange(4, dtype=jnp.int32))
  return out


# speedup vs baseline: 14.6832x; 1.5137x over previous
import functools

import jax
import jax.numpy as jnp
from jax import lax
from jax.experimental import pallas as pl
from jax.experimental.pallas import tpu as pltpu
from jax.experimental.pallas import tpu_sc as plsc

NC = 2
NS = 16
L = 16
NW = NC * NS

C = 32
MBUF = 4
NBUF = 4

N = 10000
D = 128
NPAD = 10240
YROWS = 10112
RPT = YROWS // NS

CN = 64


def _mv_body(x_hbm, rows_hbm, cols_hbm, norm_hbm, out_hbm,
             rb, cb, nb, gb, csc, y_sh, msems, gsems, ssems,
             *, nchunk):
  cid = lax.axis_index("c")
  sid = lax.axis_index("s")
  wid = cid * NS + sid
  ebase = wid * (nchunk * C)

  zv = jnp.zeros((L,), jnp.float32)

  def zrow(r, _):
    for j in range(D // L):
      gb[0][r, pl.ds(j * L, L)] = zv
    return 0

  lax.fori_loop(0, C, zrow, 0)
  base = sid * RPT
  for k in range(RPT // C):
    pltpu.sync_copy(gb[0], y_sh.at[pl.ds(base + k * C, C)])
  rem = RPT % C
  if rem:
    pltpu.sync_copy(gb[0].at[pl.ds(0, rem)],
                    y_sh.at[pl.ds(base + (RPT // C) * C, rem)])
  plsc.subcore_barrier()

  def meta_issue(i, m):
    off = ebase + i * C
    pltpu.async_copy(rows_hbm.at[pl.ds(off, C)], rb[m], msems[m])
    pltpu.async_copy(cols_hbm.at[pl.ds(off, C)], cb[m], msems[m])
    pltpu.async_copy(norm_hbm.at[pl.ds(off, C)], nb[m], msems[m])

  def meta_wait(m):
    pltpu.make_async_copy(rows_hbm.at[pl.ds(0, C)], rb[m], msems[m]).wait()
    pltpu.make_async_copy(cols_hbm.at[pl.ds(0, C)], cb[m], msems[m]).wait()
    pltpu.make_async_copy(norm_hbm.at[pl.ds(0, C)], nb[m], msems[m]).wait()

  def gather_issue(q):
    pltpu.async_copy(x_hbm.at[rb[q]], gb[q], gsems[q])

  def gather_wait(q):
    pltpu.make_async_copy(x_hbm.at[rb[0]], gb[q], gsems[q]).wait()

  def scatter_issue(q):
    pltpu.async_copy(gb[q], y_sh.at[csc.at[q]], ssems[q], add=True)

  def scatter_wait(q):
    pltpu.make_async_copy(gb[0], y_sh.at[csc.at[0]], ssems[q]).wait()

  def scale(q):
    for k in range(C // L):
      csc[q, pl.ds(k * L, L)] = cb[q][pl.ds(k * L, L)]
    for k in range(C // L):
      nv = nb[q][pl.ds(k * L, L)]
      for t in range(L):
        sv = jnp.full((L,), nv[t], jnp.float32)
        e0 = k * L + t
        for j in range(D // L):
          gb[q][e0, pl.ds(j * L, L)] = gb[q][e0, pl.ds(j * L, L)] * sv

  meta_issue(0, 0)
  meta_issue(1, 1)
  meta_issue(2, 2)
  meta_issue(3, 3)
  meta_wait(0)
  gather_issue(0)
  meta_wait(1)
  gather_issue(1)
  for i in range(2):
    meta_wait(i + 2)
    gather_issue(i + 2)
    gather_wait(i)
    scale(i)
    meta_issue(i + 4, i)
    scatter_issue(i)

  def group(g, _):
    for j in range(4):
      i = (g - 1) * 4 + 2 + j
      q = (2 + j) % 4
      qn = j % 4
      meta_wait(qn)
      scatter_wait(qn)
      gather_issue(qn)
      gather_wait(q)
      scale(q)

      @pl.when(i + 4 < nchunk)
      def _():
        meta_issue(i + 4, q)

      scatter_issue(q)
    return 0

  lax.fori_loop(1, (nchunk - 4) // 4 + 1, group, 0)

  for j in range(2):
    q = 2 + j
    scatter_wait(j)
    gather_wait(q)
    scale(q)
    scatter_issue(q)
  scatter_wait(2)
  scatter_wait(3)
  plsc.subcore_barrier()

  pltpu.sync_copy(y_sh.at[pl.ds(sid * RPT, RPT)],
                  out_hbm.at[cid, pl.ds(sid * RPT, RPT)])


def _norm_body(rows_hbm, cols_hbm, norm_hbm,
               rows_v, cols_v, wbuf, deg_v, dinv_v, deg_sh, dinv_sh, dsem,
               *, nchunk):
  cid = lax.axis_index("c")
  sid = lax.axis_index("s")
  wid = cid * NS + sid
  tpe = nchunk * CN
  seg = NPAD // NS

  zv = jnp.zeros((L,), jnp.float32)

  def zdeg(i, _):
    deg_v[pl.ds(i * L, L)] = zv
    return 0

  lax.fori_loop(0, NPAD // L, zdeg, 0)
  pltpu.sync_copy(deg_v.at[pl.ds(0, seg)], deg_sh.at[pl.ds(sid * seg, seg)])
  plsc.subcore_barrier()

  for t in range(2):
    wt = sid * 2 + t
    pltpu.sync_copy(rows_hbm.at[wt], rows_v)
    pltpu.sync_copy(cols_hbm.at[wt], cols_v)

    def wrow(ci, _):
      for k in range(CN // L):
        r = rows_v[ci, pl.ds(k * L, L)]
        c = cols_v[ci, pl.ds(k * L, L)]
        wbuf[ci, pl.ds(k * L, L)] = jnp.where(r != c, 1.0, 0.0)
      return 0

    lax.fori_loop(0, nchunk, wrow, 0)

    def wscat(ci, _):
      pltpu.async_copy(wbuf.at[ci], deg_sh.at[rows_v.at[ci]], dsem, add=True)
      return 0

    lax.fori_loop(0, nchunk, wscat, 0)

    def wdrain(ci, _):
      pltpu.make_async_copy(wbuf.at[0], deg_sh.at[rows_v.at[0]], dsem).wait()
      return 0

    lax.fori_loop(0, nchunk, wdrain, 0)

  plsc.subcore_barrier()

  pltpu.sync_copy(deg_sh.at[pl.ds(sid * seg, seg)], deg_v.at[pl.ds(0, seg)])

  def rsq(i, _):
    d = deg_v[pl.ds(i * L, L)]
    ds = jnp.maximum(d, 1.0)
    y = jnp.full((L,), 0.015625, jnp.float32)
    for _ in range(16):
      y = y * (1.5 - 0.5 * ds * y * y)
    dinv_v[pl.ds(i * L, L)] = jnp.where(d > 0.5, y, 0.0)
    return 0

  lax.fori_loop(0, seg // L, rsq, 0)
  pltpu.sync_copy(dinv_v.at[pl.ds(0, seg)], dinv_sh.at[pl.ds(sid * seg, seg)])
  plsc.subcore_barrier()
  pltpu.sync_copy(dinv_sh, dinv_v)

  pltpu.sync_copy(rows_hbm.at[wid], rows_v)
  pltpu.sync_copy(cols_hbm.at[wid], cols_v)

  def ncomp(ci, _):
    for k in range(CN // L):
      r = rows_v[ci, pl.ds(k * L, L)]
      c = cols_v[ci, pl.ds(k * L, L)]
      dr = plsc.load_gather(dinv_v, [r])
      dc = plsc.load_gather(dinv_v, [c])
      wbuf[ci, pl.ds(k * L, L)] = jnp.where(r != c, -(dr * dc), 0.0)
    return 0

  lax.fori_loop(0, nchunk, ncomp, 0)

  def nout(ci, _):
    pltpu.async_copy(wbuf.at[ci], norm_hbm.at[pl.ds(wid * tpe + ci * CN, CN)],
                     dsem)
    return 0

  lax.fori_loop(0, nchunk, nout, 0)

  def ndrain(ci, _):
    pltpu.make_async_copy(wbuf.at[0],
                          norm_hbm.at[pl.ds(0, CN)], dsem).wait()
    return 0

  lax.fori_loop(0, nchunk, ndrain, 0)


def _make_sc_kernels(nchunk, nchunk_n, e_pad):
  mesh = plsc.VectorSubcoreMesh(core_axis_name="c", subcore_axis_name="s")
  params = pltpu.CompilerParams(needs_layout_passes=False)

  mv = pl.kernel(
      functools.partial(_mv_body, nchunk=nchunk),
      out_type=jax.ShapeDtypeStruct((NC, YROWS, D), jnp.float32),
      mesh=mesh,
      scratch_types=[
          [pltpu.VMEM((C,), jnp.int32) for _ in range(MBUF)],
          [pltpu.VMEM((C,), jnp.int32) for _ in range(MBUF)],
          [pltpu.VMEM((C,), jnp.float32) for _ in range(MBUF)],
          [pltpu.VMEM((C, D), jnp.float32) for _ in range(NBUF)],
          pltpu.VMEM((NBUF, C), jnp.int32),
          pltpu.VMEM_SHARED((YROWS, D), jnp.float32),
          [pltpu.SemaphoreType.DMA for _ in range(MBUF)],
          [pltpu.SemaphoreType.DMA for _ in range(NBUF)],
          [pltpu.SemaphoreType.DMA for _ in range(NBUF)],
      ],
      compiler_params=params,
  )

  norm_k = pl.kernel(
      functools.partial(_norm_body, nchunk=nchunk_n),
      out_type=jax.ShapeDtypeStruct((e_pad,), jnp.float32),
      mesh=mesh,
      scratch_types=[
          pltpu.VMEM((nchunk_n, CN), jnp.int32),
          pltpu.VMEM((nchunk_n, CN), jnp.int32),
          pltpu.VMEM((nchunk_n, CN), jnp.float32),
          pltpu.VMEM((NPAD,), jnp.float32),
          pltpu.VMEM((NPAD,), jnp.float32),
          pltpu.VMEM_SHARED((NPAD,), jnp.float32),
          pltpu.VMEM_SHARED((NPAD,), jnp.float32),
          pltpu.SemaphoreType.DMA,
      ],
      compiler_params=params,
  )
  return mv, norm_k


RBLK = 400


def _init_body(x_ref, w0_ref, b_ref, out_ref):
  out_ref[...] = (
      jnp.dot(x_ref[...], w0_ref[...], preferred_element_type=jnp.float32)
      + b_ref[...])


def _init(x, w0, b2d):
  return pl.pallas_call(
      _init_body,
      grid=(N // RBLK,),
      in_specs=[
          pl.BlockSpec((RBLK, D), lambda i: (i, 0)),
          pl.BlockSpec((D, D), lambda i: (0, 0)),
          pl.BlockSpec((1, D), lambda i: (0, 0)),
      ],
      out_specs=pl.BlockSpec((RBLK, D), lambda i: (i, 0)),
      out_shape=jax.ShapeDtypeStruct((N, D), jnp.float32),
  )(x, w0, b2d)


def _comb_body(yp_ref, prev_ref, acc_ref, wk_ref, ab_ref, tx_ref, out_ref):
  a = ab_ref[0, 0]
  be = ab_ref[0, 1]
  tx = a * (yp_ref[0] + yp_ref[1]) - be * prev_ref[...]
  tx_ref[...] = tx
  out_ref[...] = acc_ref[...] + jnp.dot(
      tx, wk_ref[0], preferred_element_type=jnp.float32)


def _comb(yp, prev, acc, wk, ab):
  return pl.pallas_call(
      _comb_body,
      grid=(N // RBLK,),
      in_specs=[
          pl.BlockSpec((NC, RBLK, D), lambda i: (0, i, 0)),
          pl.BlockSpec((RBLK, D), lambda i: (i, 0)),
          pl.BlockSpec((RBLK, D), lambda i: (i, 0)),
          pl.BlockSpec((1, D, D), lambda i: (0, 0, 0)),
          pl.BlockSpec((1, 2), lambda i: (0, 0)),
      ],
      out_specs=[
          pl.BlockSpec((RBLK, D), lambda i: (i, 0)),
          pl.BlockSpec((RBLK, D), lambda i: (i, 0)),
      ],
      out_shape=[
          jax.ShapeDtypeStruct((N, D), jnp.float32),
          jax.ShapeDtypeStruct((N, D), jnp.float32),
      ],
  )(yp, prev, acc, wk, ab)


def kernel(x_node, edge_index, W, b):
  n, d = x_node.shape
  e = edge_index.shape[0]
  assert (n, d) == (N, D)

  tpe = -(-e // NW)
  tpe = -(-tpe // 128) * 128
  nchunk = tpe // C
  nchunk_n = tpe // CN
  e_pad = tpe * NW

  row = edge_index[:, 0]
  col = edge_index[:, 1]
  pad = jnp.arange(e_pad - e, dtype=jnp.int32) % N
  rows1 = jnp.concatenate([row, pad])
  cols1 = jnp.concatenate([col, pad])
  rows3 = rows1.reshape(NW, nchunk_n, CN)
  cols3 = cols1.reshape(NW, nchunk_n, CN)

  mv, norm_k = _make_sc_kernels(nchunk, nchunk_n, e_pad)

  norm1 = norm_k(rows3, cols3)
  b2d = b.reshape(1, D)
  out0 = _init(x_node, W[0], b2d)

  txm1, txm2, out = x_node, x_node, out0
  for i in range(4):
    yp = mv(txm1, rows1, cols1, norm1)
    ab = jnp.array([[1.0 if i == 0 else 2.0,
                     0.0 if i == 0 else 1.0]], dtype=jnp.float32)
    tx, out = _comb(yp, txm2, out, W[i + 1:i + 2], ab)
    txm1, txm2 = tx, txm1
  return out
